# baseline (speedup 1.0000x reference)
import functools

import jax
import jax.numpy as jnp
from jax import lax
from jax.experimental import pallas as pl
from jax.experimental.pallas import tpu as pltpu
from jax.experimental.pallas import tpu_sc as plsc

_CHUNK = 96


def _build_gather(N, D, dtype):
    info = plsc.get_sparse_core_info()
    NC, NS = info.num_cores, info.num_subcores
    NW = NC * NS
    rpw = N // NW
    assert N % NW == 0 and rpw % _CHUNK == 0

    mesh = plsc.VectorSubcoreMesh(core_axis_name="c", subcore_axis_name="s")

    @functools.partial(
        pl.kernel,
        mesh=mesh,
        out_type=jax.ShapeDtypeStruct((N, D), dtype),
        compiler_params=pltpu.CompilerParams(use_tc_tiling_on_sc=False),
        scratch_types=[
            pltpu.VMEM((rpw,), jnp.int32),
            pltpu.VMEM((rpw, D), dtype),
            pltpu.SemaphoreType.DMA,
        ],
    )
    def k(tab_hbm, idx_hbm, tags_hbm, idx_v, t_v, sem_g):
        wid = lax.axis_index("s") * NC + lax.axis_index("c")
        base = wid * rpw
        pltpu.sync_copy(idx_hbm.at[pl.ds(base, rpw)], idx_v)

        def fire(c, carry):
            pltpu.async_copy(
                tab_hbm.at[idx_v.at[pl.ds(c * _CHUNK, _CHUNK)]],
                t_v.at[pl.ds(c * _CHUNK, _CHUNK)], sem_g)
            return carry

        lax.fori_loop(0, rpw // _CHUNK, fire, 0)
        pltpu.make_async_copy(tab_hbm.at[pl.ds(0, rpw)], t_v, sem_g).wait()
        pltpu.sync_copy(t_v, tags_hbm.at[pl.ds(base, rpw)])

    return k


def _repack_rowmajor(tag_table, DW):
    V, D = tag_table.shape
    dt = tag_table.dtype
    pad_eye = jnp.concatenate(
        [jnp.eye(D, dtype=dt), jnp.zeros((D, DW - D), dtype=dt)], axis=1)
    return tag_table @ pad_eye


def kernel(global_feat, region_feats, tag_table, labels):
    B, R, D = region_feats.shape
    V = tag_table.shape[0]
    tab = jnp.concatenate(
        [tag_table[0::2], tag_table[1::2]], axis=1).reshape(V, D)
    idx = labels.reshape(-1)
    k = _build_gather(B * R, D, region_feats.dtype)
    tags = k(tab, idx).reshape(B, R, D)
    return jnp.concatenate([global_feat, region_feats, tags], axis=1)


# speedup vs baseline: 29.2091x; 29.2091x over previous
import functools

import jax
import jax.numpy as jnp
from jax import lax
from jax.experimental import pallas as pl
from jax.experimental.pallas import tpu as pltpu
from jax.experimental.pallas import tpu_sc as plsc

_CHUNK = 128


def _build_gather(N, D, dtype):
    info = plsc.get_sparse_core_info()
    NC, NS = info.num_cores, info.num_subcores
    NW = NC * NS
    rpw = N // NW
    assert N % NW == 0 and rpw % _CHUNK == 0

    mesh = plsc.VectorSubcoreMesh(core_axis_name="c", subcore_axis_name="s")

    @functools.partial(
        pl.kernel,
        mesh=mesh,
        out_type=jax.ShapeDtypeStruct((N, D), dtype),
        compiler_params=pltpu.CompilerParams(use_tc_tiling_on_sc=False),
        scratch_types=[
            pltpu.VMEM((rpw,), jnp.int32),
            pltpu.VMEM((rpw, D), dtype),
            pltpu.SemaphoreType.DMA,
        ],
    )
    def k(tab_hbm, idx_hbm, tags_hbm, idx_v, t_v, sem_g):
        wid = lax.axis_index("s") * NC + lax.axis_index("c")
        base = wid * rpw
        pltpu.sync_copy(idx_hbm.at[pl.ds(base, rpw)], idx_v)

        def fire(c, carry):
            pltpu.async_copy(
                tab_hbm.at[idx_v.at[pl.ds(c * _CHUNK, _CHUNK)]],
                t_v.at[pl.ds(c * _CHUNK, _CHUNK)], sem_g)
            return carry

        lax.fori_loop(0, rpw // _CHUNK, fire, 0)
        pltpu.make_async_copy(tab_hbm.at[pl.ds(0, rpw)], t_v, sem_g).wait()
        pltpu.sync_copy(t_v, tags_hbm.at[pl.ds(base, rpw)])

    return k


def _repack_rowmajor(tag_table, DW):
    V, D = tag_table.shape
    dt = tag_table.dtype
    pad_eye = jnp.concatenate(
        [jnp.eye(D, dtype=dt), jnp.zeros((D, DW - D), dtype=dt)], axis=1)
    return tag_table @ pad_eye


def kernel(global_feat, region_feats, tag_table, labels):
    B, R, D = region_feats.shape
    V = tag_table.shape[0]
    tab = _repack_rowmajor(tag_table, 2 * D).reshape(2 * V, D)
    idx2 = labels.reshape(-1) * 2
    k = _build_gather(B * R, D, region_feats.dtype)
    tags = k(tab, idx2).reshape(B, R, D)
    return jnp.concatenate([global_feat, region_feats, tags], axis=1)
